# baseline (device time: 29940 ns/iter reference)
import jax
import jax.numpy as jnp
from jax import lax
from jax.experimental import pallas as pl
from jax.experimental.pallas import tpu as pltpu

T = 512
D = 1024
V_LOCAL = 8192
NCHUNK = 16
CB = V_LOCAL // NCHUNK


def kernel(x, W, labels):
    labels2 = labels.reshape(T, 1)

    def body(
        x_ref, w_hbm, lab_ref, out_ref,
        w_vmem, acc_ref, peer_ref, copy_sems, send_sem, recv_sem,
    ):
        my_x = lax.axis_index("x")
        my_y = lax.axis_index("y")
        my_z = lax.axis_index("z")
        partner = (my_x, my_y, 1 - my_z)

        copies = []
        for c in range(NCHUNK):
            cp = pltpu.make_async_copy(
                w_hbm.at[:, pl.ds(c * CB, CB)],
                w_vmem.at[:, pl.ds(c * CB, CB)],
                copy_sems.at[c],
            )
            cp.start()
            copies.append(cp)

        barrier = pltpu.get_barrier_semaphore()
        pl.semaphore_signal(
            barrier, inc=1, device_id=partner, device_id_type=pl.DeviceIdType.MESH
        )

        xb = x_ref[...].astype(jnp.bfloat16)
        lab = lab_ref[...]
        s_tot = jnp.zeros((T, 1), jnp.float32)
        lab_tot = jnp.zeros((T, 1), jnp.float32)
        for c in range(NCHUNK):
            copies[c].wait()
            wb = w_vmem[:, c * CB:(c + 1) * CB].astype(jnp.bfloat16)
            logits = jnp.dot(xb, wb, preferred_element_type=jnp.float32)
            s_tot += jnp.sum(jnp.exp(logits), axis=1, keepdims=True)
            ids = lax.broadcasted_iota(jnp.int32, (T, CB), 1) + (
                c * CB + my_z * V_LOCAL
            )
            lab_tot += jnp.sum(
                jnp.where(ids == lab, logits, 0.0), axis=1, keepdims=True
            )
        acc_ref[...] = jnp.concatenate([s_tot, lab_tot], axis=1)

        pl.semaphore_wait(barrier, 1)
        rdma = pltpu.make_async_remote_copy(
            src_ref=acc_ref,
            dst_ref=peer_ref,
            send_sem=send_sem,
            recv_sem=recv_sem,
            device_id=partner,
            device_id_type=pl.DeviceIdType.MESH,
        )
        rdma.start()
        rdma.wait()
        s = acc_ref[:, 0:1] + peer_ref[:, 0:1]
        lab_logit = acc_ref[:, 1:2] + peer_ref[:, 1:2]
        out_ref[...] = jnp.log(s) - lab_logit

    out = pl.pallas_call(
        body,
        out_shape=jax.ShapeDtypeStruct((T, 1), jnp.float32),
        in_specs=[
            pl.BlockSpec(memory_space=pltpu.VMEM),
            pl.BlockSpec(memory_space=pl.ANY),
            pl.BlockSpec(memory_space=pltpu.VMEM),
        ],
        out_specs=pl.BlockSpec(memory_space=pltpu.VMEM),
        scratch_shapes=[
            pltpu.VMEM((D, V_LOCAL), jnp.float32),
            pltpu.VMEM((T, 2), jnp.float32),
            pltpu.VMEM((T, 2), jnp.float32),
            pltpu.SemaphoreType.DMA((NCHUNK,)),
            pltpu.SemaphoreType.DMA,
            pltpu.SemaphoreType.DMA,
        ],
        compiler_params=pltpu.CompilerParams(
            collective_id=0, vmem_limit_bytes=100 * 1024 * 1024
        ),
    )(x, W, labels2)
    return out.reshape(T)
